# Initial kernel scaffold; baseline (speedup 1.0000x reference)
import jax
import jax.numpy as jnp
from jax.experimental import pallas as pl


def kernel(xyz, query, value, neigh_idx, idx_base, W_lpe, bn_gamma, bn_beta, bn_mean, bn_var):
    raise NotImplementedError("write your pallas kernel here")


# speedup vs baseline: 5.9534x; 5.9534x over previous
import jax
import jax.numpy as jnp
from jax import lax
from jax.experimental import pallas as pl
from jax.experimental.pallas import tpu as pltpu
from jax.experimental.pallas import tpu_sc as plsc

B = 4
N = 4096
K = 16
G = 4
CQ = 64
CV = 64
CQG = CQ // G
CVG = CV // G
DTBL = 144
DOWN = 68
NC = 2
NS = 16
PTS_PER_TILE = (B * N) // (NC * NS)
CHUNK = 16
NCHUNK = PTS_PER_TILE // CHUNK
MCOL = N // 8


def _rsqrt(s):
    i = plsc.bitcast(s, jnp.int32)
    i = jnp.int32(0x5F3759DF) - (i >> 1)
    y = plsc.bitcast(i, jnp.float32)
    for _ in range(3):
        y = y * (1.5 - 0.5 * s * y * y)
    return y


def _sc_body(tbl, otbl, nidx_hbm, base_hbm, params_hbm,
             lv_hbm, fx_hbm, cent_hbm, cpart_hbm,
             nbA, nbB, gidxA, gidxB, ownA, ownB, fxA, fxB, lvA, lvB,
             nidx_all, la_buf, ski_buf, cent_buf, params_v, base_v,
             macc, mbuf,
             semgA, semgB, semoA, semoB, semwA, semwB):
    c_idx = lax.axis_index("c")
    s_idx = lax.axis_index("s")
    bl = s_idx // 8
    b = 2 * c_idx + bl
    part = s_idx % 8
    n0 = part * PTS_PER_TILE
    p0 = b * N + n0

    iota = lax.iota(jnp.int32, 16)

    def gather_descs(nb, gidx, semg):
        return (pltpu.make_async_copy(tbl.at[gidx.at[0]],
                                      nb.at[pl.ds(0, 128), :], semg),
                pltpu.make_async_copy(tbl.at[gidx.at[1]],
                                      nb.at[pl.ds(128, 128), :], semg))

    def own_desc(own, chunk, semo):
        pb = p0 + chunk * CHUNK
        return pltpu.make_async_copy(otbl.at[pl.ds(pb, CHUNK), :], own, semo)

    def write_descs(fx, lv, chunk, semw):
        nc = n0 + chunk * CHUNK
        ds = [pltpu.make_async_copy(
            fx, fx_hbm.at[b, g, :, pl.ds(nc, CHUNK), :], semw)
            for g in range(G)]
        ds.append(pltpu.make_async_copy(
            lv, lv_hbm.at[b, pl.ds(nc, CHUNK), :], semw))
        return ds

    def build_gidx(gidx, chunk, bsv):
        for i in range(CHUNK):
            v = nidx_all[chunk * CHUNK + i, :] + bsv
            gidx[i // 8, pl.ds((i % 8) * 16, 16)] = v

    pltpu.sync_copy(params_hbm, params_v)
    pltpu.sync_copy(base_hbm, base_v)
    pltpu.sync_copy(nidx_hbm.at[pl.ds(p0, PTS_PER_TILE), :], nidx_all)
    bsv = plsc.load_gather(base_v, [jnp.full((16,), b, jnp.int32)])

    zero16 = jnp.zeros((16,), jnp.float32)

    def zero_body(i, _):
        for g in range(G):
            cent_buf[g, pl.ds(i * 16, 16)] = zero16
        return 0
    lax.fori_loop(0, N // 16, zero_body, 0)

    build_gidx(gidxA, 0, bsv)
    for d in gather_descs(nbA, gidxA, semgA):
        d.start()
    own_desc(ownA, 0, semoA).start()
    build_gidx(gidxB, 1, bsv)
    for d in gather_descs(nbB, gidxB, semgB):
        d.start()
    own_desc(ownB, 1, semoB).start()

    wdv = params_v[0, :]
    axv = params_v[1, :]
    ayv = params_v[2, :]
    azv = params_v[3, :]
    bxv = params_v[4, :]
    byv = params_v[5, :]
    bzv = params_v[6, :]
    bbv = params_v[7, :]
    im1 = jnp.maximum(iota - 1, 0)
    ip1 = jnp.minimum(iota + 1, 15)

    def compute_chunk(nb, own, fx, lv, chunk):
        def point(pp, _):
            r0 = pp * 16
            rows = r0 + iota
            ov = own[pp, pl.ds(52, 16)]
            x0 = ov[12]
            y0 = ov[13]
            z0 = ov[14]

            las = []
            for g in range(G):
                own_q = own[pp, pl.ds(g * CQG, CQG)]
                acc = zero16
                for cc in range(CQG):
                    col = g * CQG + cc
                    lk = plsc.load_gather(
                        nb, [rows, jnp.full((16,), col, jnp.int32)])
                    acc = acc + own_q[cc] * lk
                m = jnp.max(acc)
                e = jnp.exp(acc - m)
                la_g = e / jnp.broadcast_to(jnp.sum(e), (16,))
                la_buf[g, pp, :] = la_g
                las.append(la_g)

            xn = plsc.load_gather(nb, [rows, jnp.full((16,), 128, jnp.int32)])
            yn = plsc.load_gather(nb, [rows, jnp.full((16,), 129, jnp.int32)])
            zn = plsc.load_gather(nb, [rows, jnp.full((16,), 130, jnp.int32)])
            dx = x0 - xn
            dy = y0 - yn
            dz = z0 - zn
            s2 = dx * dx + dy * dy + dz * dz + 1e-12
            dist = s2 * _rsqrt(s2)
            base_vec = x0 * axv + y0 * ayv + z0 * azv + bbv

            accv = [zero16] * G
            accf = [zero16] * G
            ppv = jnp.full((16,), pp, jnp.int32)
            for j in range(K):
                nbx = nb[r0 + j, pl.ds(128, 16)]
                fx0j = (base_vec + dist[j] * wdv + nbx[0] * bxv
                        + nbx[1] * byv + nbx[2] * bzv)
                fx0j = jnp.maximum(fx0j, 0.0)
                plsc.store_scatter(
                    fx, [iota, ppv, jnp.full((16,), j, jnp.int32)], fx0j)
                for g in range(G):
                    laj = las[g][j]
                    vrow = nb[r0 + j, pl.ds(CQ + g * CVG, CVG)]
                    accv[g] = accv[g] + laj * vrow
                    accf[g] = accf[g] + laj * fx0j

            for g in range(G):
                lv[pp, pl.ds(g * 2 * CVG, CVG)] = accv[g]
                lv[pp, pl.ds(g * 2 * CVG + CVG, CVG)] = accf[g]

            nk = nidx_all[chunk * CHUNK + pp, :]
            for g in range(G):
                sk, sv = plsc.sort_key_val(nk, las[g])
                cum = plsc.cumsum(sv)
                la_buf[g, pp, :] = cum
                ski_buf[:] = sk
                prev_k = plsc.load_gather(ski_buf, [im1])
                next_k = plsc.load_gather(ski_buf, [ip1])
                prev_c = plsc.load_gather(la_buf.at[g, pp], [im1])
                first = (iota == 0) | (sk != prev_k)
                last = (iota == 15) | (sk != next_k)
                gfull = jnp.full((16,), g, jnp.int32)
                plsc.addupdate_scatter(cent_buf, [gfull, sk], cum, mask=last)
                plsc.addupdate_scatter(cent_buf, [gfull, sk], -prev_c,
                                       mask=first & (iota != 0))
            return 0

        lax.fori_loop(0, CHUNK, point, 0)

    def half(nb, gidx, own, fx, lv, semg, semo, semw, chunk):
        for d in gather_descs(nb, gidx, semg):
            d.wait()
        own_desc(own, chunk, semo).wait()

        @pl.when(chunk >= 2)
        def _():
            for d in write_descs(fx, lv, chunk, semw):
                d.wait()

        compute_chunk(nb, own, fx, lv, chunk)

        @pl.when(chunk + 2 < NCHUNK)
        def _():
            build_gidx(gidx, chunk + 2, bsv)
            for d in gather_descs(nb, gidx, semg):
                d.start()
            own_desc(own, chunk + 2, semo).start()

        for d in write_descs(fx, lv, chunk, semw):
            d.start()

    def step(t2, _):
        half(nbA, gidxA, ownA, fxA, lvA, semgA, semoA, semwA, 2 * t2)
        half(nbB, gidxB, ownB, fxB, lvB, semgB, semoB, semwB, 2 * t2 + 1)
        return 0

    lax.fori_loop(0, NCHUNK // 2, step, 0)

    for d in write_descs(fxA, lvA, NCHUNK - 2, semwA):
        d.wait()
    for d in write_descs(fxB, lvB, NCHUNK - 1, semwB):
        d.wait()

    pltpu.sync_copy(cent_buf, cpart_hbm.at[c_idx, s_idx])
    plsc.subcore_barrier()
    blm = s_idx // 8
    colm = (s_idx % 8) * MCOL
    pltpu.sync_copy(cpart_hbm.at[c_idx, blm * 8, :, pl.ds(colm, MCOL)], macc)
    for t in range(1, 8):
        pltpu.sync_copy(cpart_hbm.at[c_idx, blm * 8 + t, :, pl.ds(colm, MCOL)],
                        mbuf)

        def add_body(i, _):
            for g in range(G):
                sl = pl.ds(i * 16, 16)
                macc[g, sl] = macc[g, sl] + mbuf[g, sl]
            return 0
        lax.fori_loop(0, MCOL // 16, add_body, 0)
    pltpu.sync_copy(macc, cent_hbm.at[2 * c_idx + blm, :, pl.ds(colm, MCOL)])


@jax.jit
def kernel(xyz, query, value, neigh_idx, idx_base, W_lpe, bn_gamma, bn_beta,
           bn_mean, bn_var):
    f32 = jnp.float32
    qT = jnp.transpose(query, (0, 2, 1)).reshape(B * N, CQ).astype(f32)
    vT = jnp.transpose(value[..., 0], (0, 2, 1)).reshape(B * N, CV).astype(f32)
    xf = xyz.reshape(B * N, 3).astype(f32)
    zpad = jnp.zeros((B * N, DTBL - CQ - CV - 3), f32)
    tbl = jnp.concatenate([qT, vT, xf, zpad], axis=1)
    otbl = jnp.concatenate([qT, xf, jnp.zeros((B * N, 1), f32)], axis=1)
    nidx = neigh_idx.reshape(B * N, K).astype(jnp.int32)
    base = jnp.zeros((16,), jnp.int32).at[:B].set(
        idx_base.reshape(B).astype(jnp.int32))

    s = bn_gamma / jnp.sqrt(bn_var + 1e-5)
    Wp = W_lpe * s[:, None]
    bp = bn_beta - bn_mean * s
    A = Wp[:, 1:4] + Wp[:, 4:7]
    Bm = Wp[:, 7:10] - Wp[:, 1:4]
    params = jnp.stack([Wp[:, 0], A[:, 0], A[:, 1], A[:, 2],
                        Bm[:, 0], Bm[:, 1], Bm[:, 2], bp]).astype(f32)

    mesh = plsc.VectorSubcoreMesh(core_axis_name="c", subcore_axis_name="s")
    run = pl.kernel(
        _sc_body,
        out_type=(
            jax.ShapeDtypeStruct((B, N, 2 * CV), f32),
            jax.ShapeDtypeStruct((B, G, CVG, N, K), f32),
            jax.ShapeDtypeStruct((B, G, N), f32),
            jax.ShapeDtypeStruct((NC, NS, G, N), f32),
        ),
        mesh=mesh,
        compiler_params=pltpu.CompilerParams(
            needs_layout_passes=False, use_tc_tiling_on_sc=False),
        scratch_types=[
            pltpu.VMEM((256, DTBL), f32),
            pltpu.VMEM((256, DTBL), f32),
            pltpu.VMEM((2, 128), jnp.int32),
            pltpu.VMEM((2, 128), jnp.int32),
            pltpu.VMEM((CHUNK, DOWN), f32),
            pltpu.VMEM((CHUNK, DOWN), f32),
            pltpu.VMEM((CVG, CHUNK, K), f32),
            pltpu.VMEM((CVG, CHUNK, K), f32),
            pltpu.VMEM((CHUNK, 2 * CV), f32),
            pltpu.VMEM((CHUNK, 2 * CV), f32),
            pltpu.VMEM((PTS_PER_TILE, K), jnp.int32),
            pltpu.VMEM((G, CHUNK, K), f32),
            pltpu.VMEM((16,), jnp.int32),
            pltpu.VMEM((G, N), f32),
            pltpu.VMEM((8, 16), f32),
            pltpu.VMEM((16,), jnp.int32),
            pltpu.VMEM((G, MCOL), f32),
            pltpu.VMEM((G, MCOL), f32),
            pltpu.SemaphoreType.DMA,
            pltpu.SemaphoreType.DMA,
            pltpu.SemaphoreType.DMA,
            pltpu.SemaphoreType.DMA,
            pltpu.SemaphoreType.DMA,
            pltpu.SemaphoreType.DMA,
        ],
    )
    lv, fx, cent, _ = run(tbl, otbl, nidx, base, params)
    return jnp.transpose(lv, (0, 2, 1))[..., None], fx, cent


# speedup vs baseline: 8.3882x; 1.4090x over previous
import jax
import jax.numpy as jnp
from jax import lax
from jax.experimental import pallas as pl
from jax.experimental.pallas import tpu as pltpu
from jax.experimental.pallas import tpu_sc as plsc

B = 4
N = 4096
K = 16
G = 4
CQ = 64
CV = 64
CQG = CQ // G
CVG = CV // G
DTBL = 144
DOWN = 68
NC = 2
NS = 16
PTS_PER_TILE = (B * N) // (NC * NS)
CHUNK = 16
NCHUNK = PTS_PER_TILE // CHUNK
MCOL = N // 8


def _rsqrt(s):
    i = plsc.bitcast(s, jnp.int32)
    i = jnp.int32(0x5F3759DF) - (i >> 1)
    y = plsc.bitcast(i, jnp.float32)
    for _ in range(3):
        y = y * (1.5 - 0.5 * s * y * y)
    return y


def _sc_body(tbl, nidx_hbm, base_hbm, params_hbm,
             lv_hbm, fx_hbm, cent_hbm, cpart_hbm,
             nbA, nbB, gidxA, gidxB, ownA, ownB, fxA, fxB, lvA, lvB,
             nidx_all, la_buf, ski_buf, cent_buf, params_v, base_v,
             macc, mbuf,
             semgA, semgB, semoA, semoB, semwA, semwB):
    c_idx = lax.axis_index("c")
    s_idx = lax.axis_index("s")
    bl = s_idx // 8
    b = 2 * c_idx + bl
    part = s_idx % 8
    n0 = part * PTS_PER_TILE
    p0 = b * N + n0

    iota = lax.iota(jnp.int32, 16)

    def gather_descs(nb, gidx, semg):
        return (pltpu.make_async_copy(tbl.at[gidx.at[0]],
                                      nb.at[pl.ds(0, 128), :], semg),
                pltpu.make_async_copy(tbl.at[gidx.at[1]],
                                      nb.at[pl.ds(128, 128), :], semg))

    def own_desc(own, chunk, semo):
        pb = p0 + chunk * CHUNK
        return pltpu.make_async_copy(tbl.at[pl.ds(pb, CHUNK), :], own, semo)

    def write_descs(fx, lv, chunk, semw):
        nc = n0 + chunk * CHUNK
        return [
            pltpu.make_async_copy(
                fx, fx_hbm.at[b, :, pl.ds(nc, CHUNK), :], semw),
            pltpu.make_async_copy(
                lv, lv_hbm.at[b, :, pl.ds(nc, CHUNK)], semw),
        ]

    def build_gidx(gidx, chunk, bsv):
        for i in range(CHUNK):
            v = nidx_all[chunk * CHUNK + i, :] + bsv
            gidx[i // 8, pl.ds((i % 8) * 16, 16)] = v

    pltpu.sync_copy(params_hbm, params_v)
    pltpu.sync_copy(base_hbm, base_v)
    pltpu.sync_copy(nidx_hbm.at[pl.ds(p0, PTS_PER_TILE), :], nidx_all)
    bsv = plsc.load_gather(base_v, [jnp.full((16,), b, jnp.int32)])

    zero16 = jnp.zeros((16,), jnp.float32)

    def zero_body(i, _):
        for g in range(G):
            cent_buf[g, pl.ds(i * 16, 16)] = zero16
        return 0
    lax.fori_loop(0, N // 16, zero_body, 0)

    build_gidx(gidxA, 0, bsv)
    for d in gather_descs(nbA, gidxA, semgA):
        d.start()
    own_desc(ownA, 0, semoA).start()
    build_gidx(gidxB, 1, bsv)
    for d in gather_descs(nbB, gidxB, semgB):
        d.start()
    own_desc(ownB, 1, semoB).start()

    wdv = params_v[0, :]
    axv = params_v[1, :]
    ayv = params_v[2, :]
    azv = params_v[3, :]
    bxv = params_v[4, :]
    byv = params_v[5, :]
    bzv = params_v[6, :]
    bbv = params_v[7, :]
    im1 = jnp.maximum(iota - 1, 0)
    ip1 = jnp.minimum(iota + 1, 15)

    def compute_chunk(nb, own, fx, lv, chunk):
        def point(pp, _):
            r0 = pp * 16
            rows = r0 + iota
            ov = own[pp, pl.ds(120, 16)]
            x0 = ov[8]
            y0 = ov[9]
            z0 = ov[10]

            las = []
            for g in range(G):
                own_q = own[pp, pl.ds(g * CQG, CQG)]
                acc = zero16
                for cc in range(CQG):
                    col = g * CQG + cc
                    lk = plsc.load_gather(
                        nb, [rows, jnp.full((16,), col, jnp.int32)])
                    acc = acc + own_q[cc] * lk
                m = jnp.max(acc)
                e = jnp.exp(acc - m)
                la_g = e / jnp.broadcast_to(jnp.sum(e), (16,))
                la_buf[g, pp, :] = la_g
                las.append(la_g)

            xn = plsc.load_gather(nb, [rows, jnp.full((16,), 128, jnp.int32)])
            yn = plsc.load_gather(nb, [rows, jnp.full((16,), 129, jnp.int32)])
            zn = plsc.load_gather(nb, [rows, jnp.full((16,), 130, jnp.int32)])
            dx = x0 - xn
            dy = y0 - yn
            dz = z0 - zn
            s2 = dx * dx + dy * dy + dz * dz + 1e-12
            dist = s2 * _rsqrt(s2)
            base_vec = x0 * axv + y0 * ayv + z0 * azv + bbv

            accv = [zero16] * G
            accf = [zero16] * G
            ppv = jnp.full((16,), pp, jnp.int32)
            for j in range(K):
                nbx = nb[r0 + j, pl.ds(128, 16)]
                fx0j = (base_vec + dist[j] * wdv + nbx[0] * bxv
                        + nbx[1] * byv + nbx[2] * bzv)
                fx0j = jnp.maximum(fx0j, 0.0)
                plsc.store_scatter(
                    fx, [iota, ppv, jnp.full((16,), j, jnp.int32)], fx0j)
                for g in range(G):
                    laj = las[g][j]
                    vrow = nb[r0 + j, pl.ds(CQ + g * CVG, CVG)]
                    accv[g] = accv[g] + laj * vrow
                    accf[g] = accf[g] + laj * fx0j

            for g in range(G):
                plsc.store_scatter(lv, [g * 2 * CVG + iota, ppv], accv[g])
                plsc.store_scatter(lv, [g * 2 * CVG + CVG + iota, ppv],
                                   accf[g])

            nk = nidx_all[chunk * CHUNK + pp, :]
            for g in range(G):
                sk, sv = plsc.sort_key_val(nk, las[g])
                cum = plsc.cumsum(sv)
                la_buf[g, pp, :] = cum
                ski_buf[:] = sk
                prev_k = plsc.load_gather(ski_buf, [im1])
                next_k = plsc.load_gather(ski_buf, [ip1])
                prev_c = plsc.load_gather(la_buf.at[g, pp], [im1])
                first = (iota == 0) | (sk != prev_k)
                last = (iota == 15) | (sk != next_k)
                gfull = jnp.full((16,), g, jnp.int32)
                plsc.addupdate_scatter(cent_buf, [gfull, sk], cum, mask=last)
                plsc.addupdate_scatter(cent_buf, [gfull, sk], -prev_c,
                                       mask=first & (iota != 0))
            return 0

        lax.fori_loop(0, CHUNK, point, 0)

    def half(nb, gidx, own, fx, lv, semg, semo, semw, chunk):
        for d in gather_descs(nb, gidx, semg):
            d.wait()
        own_desc(own, chunk, semo).wait()

        @pl.when(chunk >= 2)
        def _():
            for d in write_descs(fx, lv, chunk, semw):
                d.wait()

        compute_chunk(nb, own, fx, lv, chunk)

        @pl.when(chunk + 2 < NCHUNK)
        def _():
            build_gidx(gidx, chunk + 2, bsv)
            for d in gather_descs(nb, gidx, semg):
                d.start()
            own_desc(own, chunk + 2, semo).start()

        for d in write_descs(fx, lv, chunk, semw):
            d.start()

    def step(t2, _):
        half(nbA, gidxA, ownA, fxA, lvA, semgA, semoA, semwA, 2 * t2)
        half(nbB, gidxB, ownB, fxB, lvB, semgB, semoB, semwB, 2 * t2 + 1)
        return 0

    lax.fori_loop(0, NCHUNK // 2, step, 0)

    for d in write_descs(fxA, lvA, NCHUNK - 2, semwA):
        d.wait()
    for d in write_descs(fxB, lvB, NCHUNK - 1, semwB):
        d.wait()

    pltpu.sync_copy(cent_buf, cpart_hbm.at[c_idx, s_idx])
    plsc.subcore_barrier()
    blm = s_idx // 8
    colm = (s_idx % 8) * MCOL
    pltpu.sync_copy(cpart_hbm.at[c_idx, blm * 8, :, pl.ds(colm, MCOL)], macc)
    for t in range(1, 8):
        pltpu.sync_copy(cpart_hbm.at[c_idx, blm * 8 + t, :, pl.ds(colm, MCOL)],
                        mbuf)

        def add_body(i, _):
            for g in range(G):
                sl = pl.ds(i * 16, 16)
                macc[g, sl] = macc[g, sl] + mbuf[g, sl]
            return 0
        lax.fori_loop(0, MCOL // 16, add_body, 0)
    pltpu.sync_copy(macc, cent_hbm.at[2 * c_idx + blm, :, pl.ds(colm, MCOL)])


@jax.jit
def kernel(xyz, query, value, neigh_idx, idx_base, W_lpe, bn_gamma, bn_beta,
           bn_mean, bn_var):
    f32 = jnp.float32
    qT = jnp.transpose(query, (0, 2, 1)).reshape(B * N, CQ).astype(f32)
    vT = jnp.transpose(value[..., 0], (0, 2, 1)).reshape(B * N, CV).astype(f32)
    xf = xyz.reshape(B * N, 3).astype(f32)
    zpad = jnp.zeros((B * N, DTBL - CQ - CV - 3), f32)
    tbl = jnp.concatenate([qT, vT, xf, zpad], axis=1)
    nidx = neigh_idx.reshape(B * N, K).astype(jnp.int32)
    base = jnp.zeros((16,), jnp.int32).at[:B].set(
        idx_base.reshape(B).astype(jnp.int32))

    s = bn_gamma / jnp.sqrt(bn_var + 1e-5)
    Wp = W_lpe * s[:, None]
    bp = bn_beta - bn_mean * s
    A = Wp[:, 1:4] + Wp[:, 4:7]
    Bm = Wp[:, 7:10] - Wp[:, 1:4]
    params = jnp.stack([Wp[:, 0], A[:, 0], A[:, 1], A[:, 2],
                        Bm[:, 0], Bm[:, 1], Bm[:, 2], bp]).astype(f32)

    mesh = plsc.VectorSubcoreMesh(core_axis_name="c", subcore_axis_name="s")
    run = pl.kernel(
        _sc_body,
        out_type=(
            jax.ShapeDtypeStruct((B, 2 * CV, N), f32),
            jax.ShapeDtypeStruct((B, CVG, N, K), f32),
            jax.ShapeDtypeStruct((B, G, N), f32),
            jax.ShapeDtypeStruct((NC, NS, G, N), f32),
        ),
        mesh=mesh,
        compiler_params=pltpu.CompilerParams(
            needs_layout_passes=False, use_tc_tiling_on_sc=False),
        scratch_types=[
            pltpu.VMEM((256, DTBL), f32),
            pltpu.VMEM((256, DTBL), f32),
            pltpu.VMEM((2, 128), jnp.int32),
            pltpu.VMEM((2, 128), jnp.int32),
            pltpu.VMEM((CHUNK, DTBL), f32),
            pltpu.VMEM((CHUNK, DTBL), f32),
            pltpu.VMEM((CVG, CHUNK, K), f32),
            pltpu.VMEM((CVG, CHUNK, K), f32),
            pltpu.VMEM((2 * CV, CHUNK), f32),
            pltpu.VMEM((2 * CV, CHUNK), f32),
            pltpu.VMEM((PTS_PER_TILE, K), jnp.int32),
            pltpu.VMEM((G, CHUNK, K), f32),
            pltpu.VMEM((16,), jnp.int32),
            pltpu.VMEM((G, N), f32),
            pltpu.VMEM((8, 16), f32),
            pltpu.VMEM((16,), jnp.int32),
            pltpu.VMEM((G, MCOL), f32),
            pltpu.VMEM((G, MCOL), f32),
            pltpu.SemaphoreType.DMA,
            pltpu.SemaphoreType.DMA,
            pltpu.SemaphoreType.DMA,
            pltpu.SemaphoreType.DMA,
            pltpu.SemaphoreType.DMA,
            pltpu.SemaphoreType.DMA,
        ],
    )
    lv, fx0, cent, _ = run(tbl, nidx, base, params)
    fx = jnp.broadcast_to(fx0[:, None], (B, G, CVG, N, K))
    return lv[..., None], fx, cent
